# baseline (speedup 1.0000x reference)
import functools

import jax
import jax.numpy as jnp
from jax import lax
from jax.experimental import pallas as pl
from jax.experimental.pallas import tpu as pltpu
from jax.experimental.pallas import tpu_sc as plsc

V = 7
D = 512
B = 16384
A0 = 4096
NC = 2
NS = 16
NW = NC * NS
B_PER_W = A0 // NW
CHUNK = 64
N_CHUNKS = B_PER_W // CHUNK
NBUF = 3
COLB = D // 16

TC_BLK = 1024
TC_G = (B - A0) // TC_BLK


def _sc_lookup(idx2d, table_flat):
    mesh = plsc.VectorSubcoreMesh(core_axis_name="c", subcore_axis_name="s")

    @functools.partial(
        pl.kernel,
        mesh=mesh,
        out_type=jax.ShapeDtypeStruct((B * D,), jnp.float32),
        scratch_types=[
            pltpu.VMEM((B_PER_W,), jnp.int32),
            pltpu.VMEM((V * D,), jnp.float32),
            pltpu.VMEM((CHUNK * D,), jnp.float32),
            pltpu.VMEM((CHUNK * D,), jnp.float32),
            pltpu.VMEM((CHUNK * D,), jnp.float32),
            pltpu.SMEM((B_PER_W,), jnp.int32),
            pltpu.SemaphoreType.DMA,
            pltpu.SemaphoreType.DMA,
            pltpu.SemaphoreType.DMA,
        ],
    )
    def k(idx_hbm, tab_hbm, out_hbm, idx_v, tab_v,
          buf0, buf1, buf2, base_s, s0, s1, s2):
        wid = lax.axis_index("s") * NC + lax.axis_index("c")
        pltpu.sync_copy(tab_hbm, tab_v)
        pltpu.sync_copy(idx_hbm.at[wid], idx_v)

        for g in range(B_PER_W // 16):
            iv = idx_v[pl.ds(g * 16, 16)] * D
            for l in range(16):
                base_s[g * 16 + l] = iv[l]

        bufs = (buf0, buf1, buf2)
        ssem = (s0, s1, s2)
        sh = [None] * min(NBUF, N_CHUNKS)
        for c in range(N_CHUNKS):
            p = c % NBUF
            buf = bufs[p]
            if c >= NBUF and sh[p] is not None:
                sh[p].wait()

            @plsc.parallel_loop(0, CHUNK, unroll=4)
            def row_body(l, buf=buf, c=c):
                base = base_s[c * CHUNK + l]
                for j in range(COLB):
                    buf[pl.ds(l * D + j * 16, 16)] = tab_v[pl.ds(base + j * 16, 16)]

            sh[p] = pltpu.async_copy(
                buf,
                out_hbm.at[pl.ds((wid * B_PER_W + c * CHUNK) * D, CHUNK * D)],
                ssem[p])
        for h in sh:
            if h is not None:
                h.wait()

    return k(idx2d, table_flat)


def _tc_body(idx_ref, tab_ref, sc_ref, out_ref):
    del sc_ref
    iv = idx_ref[...]
    acc = jnp.broadcast_to(tab_ref[0][None, :], (TC_BLK, D))
    for v in range(1, V):
        acc = jnp.where(iv == v, tab_ref[v][None, :], acc)
    out_ref[...] = acc


def _tc_lookup(idx2, table8, sc_out):
    return pl.pallas_call(
        _tc_body,
        grid=(TC_G,),
        in_specs=[
            pl.BlockSpec((TC_BLK, 1), lambda i: (A0 // TC_BLK + i, 0)),
            pl.BlockSpec((8, D), lambda i: (0, 0)),
            pl.BlockSpec(memory_space=pl.ANY),
        ],
        out_specs=pl.BlockSpec((TC_BLK, D), lambda i: (A0 // TC_BLK + i, 0)),
        out_shape=jax.ShapeDtypeStruct((B, D), jnp.float32),
        input_output_aliases={2: 0},
    )(idx2, table8, sc_out)


def kernel(emotion_index, table):
    idx = emotion_index.astype(jnp.int32)
    sc_out = _sc_lookup(idx[:A0].reshape(NW, B_PER_W), table.reshape(V * D))
    table8 = jnp.concatenate(
        [table, jnp.zeros((8 - V, D), jnp.float32)], axis=0)
    return _tc_lookup(idx.reshape(B, 1), table8, sc_out.reshape(B, D))


# speedup vs baseline: 1.3748x; 1.3748x over previous
import functools

import jax
import jax.numpy as jnp
from jax import lax
from jax.experimental import pallas as pl
from jax.experimental.pallas import tpu as pltpu
from jax.experimental.pallas import tpu_sc as plsc

V = 7
D = 512
B = 16384
A0 = 4096
NC = 2
NS = 16
NW = NC * NS
B_PER_W = A0 // NW
CHUNK = 64
N_CHUNKS = B_PER_W // CHUNK
NBUF = 3
COLB = D // 16

TC_BLK = 1024
TC_G = (B - A0) // TC_BLK


def _sc_lookup(idx2d, table_flat):
    mesh = plsc.VectorSubcoreMesh(core_axis_name="c", subcore_axis_name="s")

    @functools.partial(
        pl.kernel,
        mesh=mesh,
        out_type=jax.ShapeDtypeStruct((A0 * D,), jnp.float32),
        scratch_types=[
            pltpu.VMEM((B_PER_W,), jnp.int32),
            pltpu.VMEM((V * D,), jnp.float32),
            pltpu.VMEM((CHUNK * D,), jnp.float32),
            pltpu.VMEM((CHUNK * D,), jnp.float32),
            pltpu.VMEM((CHUNK * D,), jnp.float32),
            pltpu.SMEM((B_PER_W,), jnp.int32),
            pltpu.SemaphoreType.DMA,
            pltpu.SemaphoreType.DMA,
            pltpu.SemaphoreType.DMA,
        ],
    )
    def k(idx_hbm, tab_hbm, out_hbm, idx_v, tab_v,
          buf0, buf1, buf2, base_s, s0, s1, s2):
        wid = lax.axis_index("s") * NC + lax.axis_index("c")
        pltpu.sync_copy(tab_hbm, tab_v)
        pltpu.sync_copy(idx_hbm.at[wid], idx_v)

        for g in range(B_PER_W // 16):
            iv = idx_v[pl.ds(g * 16, 16)] * D
            for l in range(16):
                base_s[g * 16 + l] = iv[l]

        bufs = (buf0, buf1, buf2)
        ssem = (s0, s1, s2)
        sh = [None] * min(NBUF, N_CHUNKS)
        for c in range(N_CHUNKS):
            p = c % NBUF
            buf = bufs[p]
            if c >= NBUF and sh[p] is not None:
                sh[p].wait()

            @plsc.parallel_loop(0, CHUNK, unroll=4)
            def row_body(l, buf=buf, c=c):
                base = base_s[c * CHUNK + l]
                for j in range(COLB):
                    buf[pl.ds(l * D + j * 16, 16)] = tab_v[pl.ds(base + j * 16, 16)]

            sh[p] = pltpu.async_copy(
                buf,
                out_hbm.at[pl.ds((wid * B_PER_W + c * CHUNK) * D, CHUNK * D)],
                ssem[p])
        for h in sh:
            if h is not None:
                h.wait()

    return k(idx2d, table_flat)


def _tc_body(idx_ref, tab_ref, out_ref):
    iv = idx_ref[...]
    acc = jnp.broadcast_to(tab_ref[0][None, :], (TC_BLK, D))
    for v in range(1, V):
        acc = jnp.where(iv == v, tab_ref[v][None, :], acc)
    out_ref[...] = acc


def _tc_lookup(idx2, table8):
    return pl.pallas_call(
        _tc_body,
        grid=(TC_G,),
        in_specs=[
            pl.BlockSpec((TC_BLK, 1), lambda i: (A0 // TC_BLK + i, 0)),
            pl.BlockSpec((8, D), lambda i: (0, 0)),
        ],
        out_specs=pl.BlockSpec((TC_BLK, D), lambda i: (A0 // TC_BLK + i, 0)),
        out_shape=jax.ShapeDtypeStruct((B, D), jnp.float32),
    )(idx2, table8)


def kernel(emotion_index, table):
    idx = emotion_index.astype(jnp.int32)
    sc_out = _sc_lookup(idx[:A0].reshape(NW, B_PER_W), table.reshape(V * D))
    table8 = jnp.concatenate(
        [table, jnp.zeros((8 - V, D), jnp.float32)], axis=0)
    tc_out = _tc_lookup(idx.reshape(B, 1), table8)
    return lax.dynamic_update_slice(tc_out, sc_out.reshape(A0, D), (0, 0))
